# Initial kernel scaffold; baseline (speedup 1.0000x reference)
import jax
import jax.numpy as jnp
from jax.experimental import pallas as pl


def kernel(points, resolution_map):
    raise NotImplementedError("write your pallas kernel here")


# speedup vs baseline: 32.4745x; 32.4745x over previous
import functools

import jax
import jax.numpy as jnp
from jax import lax
from jax.experimental import pallas as pl
from jax.experimental.pallas import tpu as pltpu
from jax.experimental.pallas import tpu_sc as plsc

NC, NS, L = 2, 16, 16
RESES = (8, 16, 32, 64)
ROW_OFF = (0, 512, 4608, 37376)
TOTAL_BINS = 299520
GRID_WORDS = TOTAL_BINS * 3
N_GROUPS = GRID_WORDS // 128
BATCHES = 8
N_POINTS = 100000
COL_TILES = 784
CT_PER_TILE = COL_TILES // NS
CHUNK_CT = 7
NCHUNK = CT_PER_TILE // CHUNK_CT
CHUNK_VREGS = CHUNK_CT * 8
RC_ROWS = 7
SLICE_WORDS = GRID_WORDS // NS
ZCHUNK = SLICE_WORDS // 27
OUT_CHUNK = 84
BATCH_PER_CORE = BATCHES // NC

_mesh = plsc.VectorSubcoreMesh(
    core_axis_name="c", subcore_axis_name="s", num_cores=NC, num_subcores=NS
)


@functools.partial(
    pl.kernel,
    out_type=jax.ShapeDtypeStruct((N_GROUPS * 8, 128), jnp.float32),
    mesh=_mesh,
    scratch_types=[
        pltpu.VMEM((3, CHUNK_CT, 8, 128), jnp.float32),
        pltpu.VMEM((12 * RC_ROWS, 128), jnp.int32),
        pltpu.VMEM((12 * RC_ROWS, 128), jnp.int32),
        pltpu.VMEM((12 * RC_ROWS, 128), jnp.float32),
        pltpu.VMEM((12 * RC_ROWS, 128), jnp.float32),
        pltpu.VMEM((ZCHUNK,), jnp.float32),
        pltpu.VMEM((6, OUT_CHUNK), jnp.int32),
        pltpu.VMEM((8, 4, 1), jnp.float32),
        pltpu.VMEM_SHARED((GRID_WORDS,), jnp.float32),
        pltpu.SemaphoreType.DMA,
        pltpu.SemaphoreType.DMA,
        pltpu.SemaphoreType.DMA,
    ],
    compiler_params=pltpu.CompilerParams(needs_layout_passes=False),
)
def _voxelize_sc(
    pts_hbm, rm_hbm, out_hbm, pts_v, idx_v0, idx_v1, val_v0, val_v1,
    zero_v, offs_v, rm_v, grid_s, sem0, sem1, sem2
):
    idxs = (idx_v0, idx_v1)
    vals_b = (val_v0, val_v1)
    core = lax.axis_index("c")
    tile = lax.axis_index("s")
    sems = (sem0, sem1)

    lane = jnp.arange(L, dtype=jnp.int32)
    zeros16 = jnp.zeros((L,), jnp.float32)

    pltpu.sync_copy(rm_hbm, rm_v)

    @pl.loop(0, ZCHUNK // L)
    def _fill(i):
        zero_v[pl.ds(i * L, L)] = zeros16

    def _zero_slice(zb):
        for h in range(27):
            pltpu.sync_copy(zero_v, grid_s.at[pl.ds(zb + h * ZCHUNK, ZCHUNK)])

    zbase = pl.multiple_of(tile * SLICE_WORDS, 32)
    _zero_slice(zbase)
    plsc.subcore_barrier()

    g0 = 439 * tile - jnp.maximum(tile - 12, 0)
    gcnt = 439 - (tile >= 12).astype(jnp.int32)

    @pl.loop(0, BATCH_PER_CORE)
    def _batch(kb):
        b = core * BATCH_PER_CORE + kb
        bvec = jnp.full((L,), b, jnp.int32)
        scales = [
            plsc.load_gather(
                rm_v,
                [bvec, jnp.full((L,), r, jnp.int32), jnp.zeros((L,), jnp.int32)],
            )
            for r in range(4)
        ]

        def _drain(pp):
            @pl.loop(0, 12 * RC_ROWS)
            def _w(q):
                pltpu.make_async_copy(
                    vals_b[pp].at[q], grid_s.at[idxs[pp].at[q]], sems[pp]
                ).wait()

        pending = [None, None]
        for ch in range(NCHUNK):
            p = ch % 2
            if pending[p] is not None:
                _drain(p)

            ct_lo = tile * CT_PER_TILE + ch * CHUNK_CT
            for c in range(3):
                pltpu.sync_copy(
                    pts_hbm.at[c, pl.ds(ct_lo, CHUNK_CT), :, :], pts_v.at[c]
                )

            @pl.loop(0, CHUNK_VREGS)
            def _vreg(j, p=p):
                ct = j // 8
                col = (j % 8) * L
                x = pts_v[0, ct, b, pl.ds(col, L)]
                y = pts_v[1, ct, b, pl.ds(col, L)]
                z = pts_v[2, ct, b, pl.ds(col, L)]
                ix = (x * 64.0).astype(jnp.int32)
                iy = (y * 64.0).astype(jnp.int32)
                iz = (z * 64.0).astype(jnp.int32)
                jj = j // 8
                vals = (x, y, z)
                for r, res in enumerate(RESES):
                    k = res.bit_length() - 1
                    sh = 6 - k
                    if sh:
                        flat = ((ix >> sh) << (2 * k)) | ((iy >> sh) << k) | (iz >> sh)
                    else:
                        flat = (ix << (2 * k)) | (iy << k) | iz
                    for ci in range(3):
                        row = (r * 3 + ci) * RC_ROWS + jj
                        idxs[p][row, pl.ds(col, L)] = (
                            flat + (ci * TOTAL_BINS + ROW_OFF[r])
                        )
                        vals_b[p][row, pl.ds(col, L)] = vals[ci] * scales[r]

            @pl.loop(0, 12 * RC_ROWS)
            def _issue(q, p=p):
                pltpu.async_copy(
                    vals_b[p].at[q], grid_s.at[idxs[p].at[q]], sems[p], add=True
                )

            pending[p] = True

        for pp in (0, 1):
            if pending[pp] is not None:
                _drain(pp)

        plsc.subcore_barrier()

        for kchunk in range(6):
            sk = g0 + jnp.minimum(84 * kchunk, gcnt - OUT_CHUNK)

            @pl.loop(0, OUT_CHUNK)
            def _stage(j, sk=sk):
                w = pl.multiple_of((sk + j) * 128, 128)
                pltpu.async_copy(grid_s.at[pl.ds(w, 128)], val_v0.at[j], sem2)

            @pl.loop(0, OUT_CHUNK)
            def _stagew(j, sk=sk):
                w = pl.multiple_of((sk + j) * 128, 128)
                pltpu.make_async_copy(
                    grid_s.at[pl.ds(w, 128)], val_v0.at[j], sem2
                ).wait()

            obase = 8 * sk + b
            for st in range(5):
                offs_v[kchunk, pl.ds(st * L, L)] = obase + 8 * (st * L + lane)
            plsc.store_scatter(
                offs_v,
                [jnp.full((L,), kchunk, jnp.int32), 80 + lane],
                obase + 8 * (80 + lane),
                mask=lane < 4,
            )
            pltpu.async_copy(
                val_v0, out_hbm.at[offs_v.at[kchunk]], sem2
            ).wait()

        plsc.subcore_barrier()
        _zero_slice(zbase)
        plsc.subcore_barrier()


def kernel(points, resolution_map):
    xp = jnp.transpose(points, (2, 0, 1))
    xp = jnp.pad(xp, ((0, 0), (0, 0), (0, COL_TILES * 128 - N_POINTS)))
    x4 = xp.reshape(3, 8, COL_TILES, 128).transpose(0, 2, 1, 3)
    out2 = _voxelize_sc(x4, resolution_map)
    return (
        out2.reshape(3, N_GROUPS // 3, 8, 128)
        .transpose(2, 1, 3, 0)
        .reshape(BATCHES, TOTAL_BINS, 3)
    )
